# Initial kernel scaffold; baseline (speedup 1.0000x reference)
import jax
import jax.numpy as jnp
from jax.experimental import pallas as pl


def kernel(feats, points_0, points_1, points_2, points_3, points_4, neighbors_0, neighbors_1, neighbors_2, neighbors_3, neighbors_4, subsampling_0, subsampling_1, subsampling_2, subsampling_3, upsampling_0, params):
    raise NotImplementedError("write your pallas kernel here")


# speedup vs baseline: 2.3174x; 2.3174x over previous
import functools

import jax
import jax.numpy as jnp
import numpy as np
from jax import lax
from jax.experimental import pallas as pl
from jax.experimental.pallas import tpu as pltpu
from jax.experimental.pallas import tpu_sc as plsc

_LRELU = 0.1
_EPS = 1e-5
_GN = 32
_RESO = 16
_KS = 15
_K = 32

_PC = pl.pallas_call


def _sc_gather(table, idx):
    V, D = table.shape
    B = idx.shape[0]
    info = plsc.get_sparse_core_info()
    nw = info.num_cores * info.num_subcores
    bpw = B // nw
    ch = min(128, bpw, 65536 // D)
    ch = 1 << (ch.bit_length() - 1)
    nch = bpw // ch
    nc = info.num_cores
    mesh = plsc.VectorSubcoreMesh(core_axis_name="c", subcore_axis_name="s")

    @functools.partial(
        pl.kernel,
        mesh=mesh,
        out_type=jax.ShapeDtypeStruct((B, D), jnp.float32),
        scratch_types=[
            pltpu.VMEM((ch,), jnp.int32),
            pltpu.VMEM((ch, D), jnp.float32),
            pltpu.SemaphoreType.DMA,
        ],
    )
    def gk(table_hbm, idx_hbm, out_hbm, idx_v, rows_v, sem):
        wid = lax.axis_index("s") * nc + lax.axis_index("c")
        base = wid * bpw

        def body(j, carry):
            off = base + j * ch
            pltpu.sync_copy(idx_hbm.at[pl.ds(off, ch)], idx_v)
            pltpu.async_copy(table_hbm.at[idx_v], rows_v, sem).wait()
            pltpu.sync_copy(rows_v, out_hbm.at[pl.ds(off, ch)])
            return carry

        lax.fori_loop(0, nch, body, 0)

    return gk(table, idx)


def _leaky(x):
    return jnp.where(x >= 0, x, _LRELU * x)


@functools.lru_cache(maxsize=None)
def _group_mats_np(c):
    cg = c // _GN
    e = np.zeros((c, _GN), np.float32)
    e[np.arange(c), np.arange(c) // cg] = 1.0
    return e, e.T.copy()


def _gn_core(y, gamma, beta, eg, egt, nrows):
    c = y.shape[1]
    cnt = float(nrows * (c // _GN))
    s1 = jnp.sum(y, axis=0, keepdims=True)
    gm = jnp.dot(s1, eg, preferred_element_type=jnp.float32) / cnt
    mean = jnp.dot(gm, egt, preferred_element_type=jnp.float32)
    y0 = y - mean
    s2 = jnp.sum(y0 * y0, axis=0, keepdims=True)
    gv = jnp.dot(s2, eg, preferred_element_type=jnp.float32) / cnt
    var = jnp.dot(gv, egt, preferred_element_type=jnp.float32)
    return y0 * lax.rsqrt(var + _EPS) * gamma + beta


def _mmgn(x, p, relu=True, sc=None):
    n = x.shape[0]
    co = p['w'].shape[1]
    eg, egt = _group_mats_np(co)
    args = [x, p['w'], p['b'].reshape(1, -1), p['g'].reshape(1, -1),
            p['be'].reshape(1, -1), jnp.asarray(eg), jnp.asarray(egt)]
    has_sc = sc is not None
    if has_sc:
        args.append(sc)

    def body(x_ref, w_ref, b_ref, g_ref, be_ref, eg_ref, egt_ref, *rest):
        if has_sc:
            sc_ref, o_ref = rest
        else:
            (o_ref,) = rest
        y = jnp.dot(x_ref[...], w_ref[...],
                    preferred_element_type=jnp.float32) + b_ref[...]
        y = _gn_core(y, g_ref[...], be_ref[...], eg_ref[...], egt_ref[...], n)
        if has_sc:
            y = y + sc_ref[...]
        if relu:
            y = _leaky(y)
        o_ref[...] = y

    return _PC(body, out_shape=jax.ShapeDtypeStruct((n, co), jnp.float32))(*args)


def _gnact(x, gamma, beta):
    n, c = x.shape
    eg, egt = _group_mats_np(c)

    def body(x_ref, g_ref, be_ref, eg_ref, egt_ref, o_ref):
        y = _gn_core(x_ref[...], g_ref[...], be_ref[...], eg_ref[...],
                     egt_ref[...], n)
        o_ref[...] = _leaky(y)

    return _PC(body, out_shape=jax.ShapeDtypeStruct((n, c), jnp.float32))(
        x, gamma.reshape(1, -1), beta.reshape(1, -1), jnp.asarray(eg),
        jnp.asarray(egt))


def _mm(x, w, b=None):
    n = x.shape[0]
    co = w.shape[1]
    has_b = b is not None
    args = [x, w] + ([b.reshape(1, -1)] if has_b else [])

    def body(x_ref, w_ref, *rest):
        if has_b:
            b_ref, o_ref = rest
        else:
            (o_ref,) = rest
        y = jnp.dot(x_ref[...], w_ref[...], preferred_element_type=jnp.float32)
        if has_b:
            y = y + b_ref[...]
        o_ref[...] = y

    return _PC(body, out_shape=jax.ShapeDtypeStruct((n, co), jnp.float32))(*args)


def _kpconv_tc(g, qrep, kpt, wmat, sigma, m):
    cp = wmat.shape[1]
    d = wmat.shape[2]
    dt = g.shape[1]
    tm = min(128, m)
    tmk = tm * _K

    def body(g_ref, q_ref, kpt_ref, w_ref, o_ref):
        gg = g_ref[...]
        rel = gg[:, 0:3] - q_ref[...]
        kp = kpt_ref[...]
        sqn = jnp.sum(rel * rel, axis=1, keepdims=True)
        dots = jnp.dot(rel, kp, preferred_element_type=jnp.float32)
        kp2 = jnp.sum(kp * kp, axis=0, keepdims=True)
        sq = sqn + kp2 - 2.0 * dots
        w = jnp.maximum(0.0, 1.0 - jnp.sqrt(sq + 1e-12) / sigma)
        f3 = gg[:, 16:16 + cp].reshape(tm, _K, cp)
        w3 = w.reshape(tm, _K, _KS)
        acc = jnp.zeros((tm, d), jnp.float32)
        for p_i in range(_KS):
            ap = jnp.sum(w3[:, :, p_i:p_i + 1] * f3, axis=1)
            acc = acc + jnp.dot(ap, w_ref[p_i],
                                preferred_element_type=jnp.float32)
        o_ref[...] = acc

    return _PC(
        body,
        grid=(m // tm,),
        in_specs=[
            pl.BlockSpec((tmk, dt), lambda i: (i, 0)),
            pl.BlockSpec((tmk, 3), lambda i: (i, 0)),
            pl.BlockSpec((3, _KS), lambda i: (0, 0)),
            pl.BlockSpec((_KS, cp, d), lambda i: (0, 0, 0)),
        ],
        out_specs=pl.BlockSpec((tm, d), lambda i: (i, 0)),
        out_shape=jax.ShapeDtypeStruct((m, d), jnp.float32),
    )(g, qrep, kpt, wmat)


def _maxred(g, m):
    c = g.shape[1]
    tm = min(128, m)
    tmk = tm * _K

    def body(g_ref, o_ref):
        f3 = g_ref[...].reshape(tm, _K, c)
        o_ref[...] = jnp.max(f3, axis=1)

    return _PC(
        body,
        grid=(m // tm,),
        in_specs=[pl.BlockSpec((tmk, c), lambda i: (i, 0))],
        out_specs=pl.BlockSpec((tm, c), lambda i: (i, 0)),
        out_shape=jax.ShapeDtypeStruct((m, c), jnp.float32),
    )(g)


def _pool_scatter(idx, cvals, nseg):
    n, c = cvals.shape

    def body(idx_ref, c_ref, o_ref):
        o_ref[...] = jnp.full((nseg, c), -jnp.inf, jnp.float32)

        def it(m_i, carry):
            seg = idx_ref[m_i]
            row = c_ref[pl.ds(m_i, 1), :]
            cur = o_ref[pl.ds(seg, 1), :]
            o_ref[pl.ds(seg, 1), :] = jnp.maximum(cur, row)
            return carry

        lax.fori_loop(0, n, it, 0)

    return _PC(
        body,
        in_specs=[
            pl.BlockSpec(memory_space=pltpu.MemorySpace.SMEM),
            pl.BlockSpec(memory_space=pltpu.MemorySpace.VMEM),
        ],
        out_specs=pl.BlockSpec(memory_space=pltpu.MemorySpace.VMEM),
        out_shape=jax.ShapeDtypeStruct((nseg, c), jnp.float32),
    )(idx, cvals)


def _nbr_table(spts, h):
    pts = jnp.concatenate([spts, jnp.full((8, 3), 1e6, jnp.float32)], 0)
    pts16 = jnp.pad(pts, ((0, 0), (0, 13)))
    fp = jnp.concatenate([h, jnp.zeros((8, h.shape[1]), jnp.float32)], 0)
    t = jnp.concatenate([pts16, fp], 1)
    dt = -t.shape[1] % 128
    if dt:
        t = jnp.pad(t, ((0, 0), (0, dt)))
    return t


def _zero_table(x):
    return jnp.concatenate([x, jnp.zeros((8, x.shape[1]), jnp.float32)], 0)


def _qrep(qp):
    m = qp.shape[0]
    return jnp.broadcast_to(qp[:, None, :], (m, _K, 3)).reshape(m * _K, 3)


def _plane_idx(p, reso):
    xy = p[:, 0:2] / (1.0 + 99.0 + 1e-3) + 0.5
    xy = jnp.clip(xy, 0.0, 1.0 - 1e-3)
    xi = (xy * reso).astype(jnp.int32)
    return xi[:, 0] + reso * xi[:, 1]


def _pool(pts_lvl, cvals, reso):
    idx = _plane_idx(pts_lvl, reso)
    plane = _pool_scatter(idx, cvals, reso * reso)
    return _sc_gather(plane, idx)


def _pad_w(kw):
    cp = max(16, kw.shape[1])
    if kw.shape[1] == cp:
        return kw
    return jnp.pad(kw, ((0, 0), (0, cp - kw.shape[1]), (0, 0)))


def _pad_feats(h):
    cp = max(16, h.shape[1])
    if h.shape[1] == cp:
        return h
    return jnp.pad(h, ((0, 0), (0, cp - h.shape[1])))


def _kpconv(h, qp, sp, neigh, kw, kp_pts, sigma):
    g = _sc_gather(_nbr_table(sp, _pad_feats(h)), neigh.reshape(-1))
    return _kpconv_tc(g, _qrep(qp), kp_pts.T, _pad_w(kw), sigma, qp.shape[0])


def _res_block(x, qp, sp, neigh, p, radius, sigma, ku, strided=False):
    h = _mmgn(x, p['u1'], relu=True)
    hp = _kpconv(h, qp, sp, neigh, p['kw'], ku * radius, sigma)
    h2 = _gnact(hp, p['g'], p['be'])
    if strided:
        scg = _sc_gather(_zero_table(x), neigh.reshape(-1))
        scx = _maxred(scg, qp.shape[0])
    else:
        scx = x
    scv = _mmgn(scx, p['sc'], relu=False) if 'sc' in p else scx
    return _mmgn(h2, p['u2'], relu=True, sc=scv)


def kernel(feats, points_0, points_1, points_2, points_3, points_4,
           neighbors_0, neighbors_1, neighbors_2, neighbors_3, neighbors_4,
           subsampling_0, subsampling_1, subsampling_2, subsampling_3,
           upsampling_0, params):
    P = params
    pts = [points_0, points_1, points_2, points_3, points_4]
    neighs = [neighbors_0, neighbors_1, neighbors_2, neighbors_3, neighbors_4]
    subs = [subsampling_0, subsampling_1, subsampling_2, subsampling_3]
    R, S = 2.0, 2.0
    ku = P['kp_unit']

    hp = _kpconv(feats, pts[0], pts[0], neighs[0], P['e11']['kw'], ku * R, S)
    x1 = _gnact(hp, P['e11']['g'], P['e11']['be'])
    x1 = _res_block(x1, pts[0], pts[0], neighs[0], P['e12'], R, S, ku)
    out1 = _mm(x1, P['l1'])
    x1 = jnp.concatenate([x1, _pool(pts[0], out1, _RESO * 8)], 1)

    x2 = _res_block(x1, pts[1], pts[0], subs[0], P['e21'], R, S, ku, True)
    x2 = _res_block(x2, pts[1], pts[1], neighs[1], P['e22'], 2 * R, 2 * S, ku)
    x2 = _res_block(x2, pts[1], pts[1], neighs[1], P['e23'], 2 * R, 2 * S, ku)
    out2 = _mm(x2, P['l2'])
    x2 = jnp.concatenate([x2, _pool(pts[1], out2, _RESO * 4)], 1)

    x3 = _res_block(x2, pts[2], pts[1], subs[1], P['e31'], 2 * R, 2 * S, ku, True)
    x3 = _res_block(x3, pts[2], pts[2], neighs[2], P['e32'], 4 * R, 4 * S, ku)
    x3 = _res_block(x3, pts[2], pts[2], neighs[2], P['e33'], 4 * R, 4 * S, ku)
    out3 = _mm(x3, P['l3'])
    x3 = jnp.concatenate([x3, _pool(pts[2], out3, _RESO * 2)], 1)

    x4 = _res_block(x3, pts[3], pts[2], subs[2], P['e41'], 4 * R, 4 * S, ku, True)
    x4 = _res_block(x4, pts[3], pts[3], neighs[3], P['e42'], 8 * R, 8 * S, ku)
    x4 = _res_block(x4, pts[3], pts[3], neighs[3], P['e43'], 8 * R, 8 * S, ku)
    out4 = _mm(x4, P['l4'])
    x4 = jnp.concatenate([x4, _pool(pts[3], out4, _RESO)], 1)

    x5 = _res_block(x4, pts[4], pts[3], subs[3], P['e51'], 8 * R, 8 * S, ku, True)
    x5 = _res_block(x5, pts[4], pts[4], neighs[4], P['e52'], 16 * R, 16 * S, ku)
    x5 = _res_block(x5, pts[4], pts[4], neighs[4], P['e53'], 16 * R, 16 * S, ku)
    feats_s5_out = _mm(x5, P['l5'])[None]

    up = _sc_gather(_zero_table(x5), upsampling_0[:, 0])
    lat4 = jnp.concatenate([up, x4], 1)
    lat4 = _mmgn(lat4, P['d40'], relu=True)
    lat4 = _mmgn(lat4, P['d41'], relu=True)
    feats_s4_out = _mm(lat4, P['d42w'], P['d42b'])[None]
    return feats_s5_out, feats_s4_out


# speedup vs baseline: 2.4680x; 1.0650x over previous
import functools

import jax
import jax.numpy as jnp
import numpy as np
from jax import lax
from jax.experimental import pallas as pl
from jax.experimental.pallas import tpu as pltpu
from jax.experimental.pallas import tpu_sc as plsc

_LRELU = 0.1
_EPS = 1e-5
_GN = 32
_RESO = 16
_KS = 15
_K = 32

_PC = pl.pallas_call


def _sc_gather(table, idx):
    V, D = table.shape
    B = idx.shape[0]
    info = plsc.get_sparse_core_info()
    nw = info.num_cores * info.num_subcores
    bpw = B // nw
    ch = min(128, bpw, 49152 // D)
    ch = 1 << (ch.bit_length() - 1)
    nch = bpw // ch
    nc = info.num_cores
    mesh = plsc.VectorSubcoreMesh(core_axis_name="c", subcore_axis_name="s")

    @functools.partial(
        pl.kernel,
        mesh=mesh,
        out_type=jax.ShapeDtypeStruct((B, D), jnp.float32),
        scratch_types=[
            pltpu.VMEM((ch,), jnp.int32),
            pltpu.VMEM((ch,), jnp.int32),
            pltpu.VMEM((ch, D), jnp.float32),
            pltpu.VMEM((ch, D), jnp.float32),
            pltpu.SemaphoreType.DMA,
            pltpu.SemaphoreType.DMA,
        ],
    )
    def gk(table_hbm, idx_hbm, out_hbm, idx0, idx1, rows0, rows1, s0, s1):
        wid = lax.axis_index("s") * nc + lax.axis_index("c")
        base = wid * bpw

        def fire(j, idx_v, rows_v, sem):
            off = base + j * ch
            pltpu.sync_copy(idx_hbm.at[pl.ds(off, ch)], idx_v)
            pltpu.async_copy(table_hbm.at[idx_v], rows_v, sem)

        def drain(j, rows_v, sem):
            off = base + j * ch
            pltpu.make_async_copy(table_hbm.at[pl.ds(0, ch)], rows_v,
                                  sem).wait()
            pltpu.sync_copy(rows_v, out_hbm.at[pl.ds(off, ch)])

        if nch == 1:
            fire(0, idx0, rows0, s0)
            drain(0, rows0, s0)
        else:
            fire(0, idx0, rows0, s0)

            def body(i, carry):
                j = 2 * i
                fire(j + 1, idx1, rows1, s1)
                drain(j, rows0, s0)
                fire(j + 2, idx0, rows0, s0)
                drain(j + 1, rows1, s1)
                return carry

            lax.fori_loop(0, nch // 2 - 1, body, 0)
            j = nch - 2
            fire(j + 1, idx1, rows1, s1)
            drain(j, rows0, s0)
            drain(j + 1, rows1, s1)

    return gk(table, idx)


def _leaky(x):
    return jnp.where(x >= 0, x, _LRELU * x)


@functools.lru_cache(maxsize=None)
def _group_mats_np(c):
    cg = c // _GN
    e = np.zeros((c, _GN), np.float32)
    e[np.arange(c), np.arange(c) // cg] = 1.0
    return e, e.T.copy()


def _gn_core(y, gamma, beta, eg, egt, nrows):
    c = y.shape[1]
    cnt = float(nrows * (c // _GN))
    s1 = jnp.sum(y, axis=0, keepdims=True)
    gm = jnp.dot(s1, eg, preferred_element_type=jnp.float32) / cnt
    mean = jnp.dot(gm, egt, preferred_element_type=jnp.float32)
    y0 = y - mean
    s2 = jnp.sum(y0 * y0, axis=0, keepdims=True)
    gv = jnp.dot(s2, eg, preferred_element_type=jnp.float32) / cnt
    var = jnp.dot(gv, egt, preferred_element_type=jnp.float32)
    return y0 * lax.rsqrt(var + _EPS) * gamma + beta


def _mmgn(x, p, relu=True, sc=None):
    n = x.shape[0]
    co = p['w'].shape[1]
    eg, egt = _group_mats_np(co)
    args = [x, p['w'], p['b'].reshape(1, -1), p['g'].reshape(1, -1),
            p['be'].reshape(1, -1), jnp.asarray(eg), jnp.asarray(egt)]
    has_sc = sc is not None
    if has_sc:
        args.append(sc)

    def body(x_ref, w_ref, b_ref, g_ref, be_ref, eg_ref, egt_ref, *rest):
        if has_sc:
            sc_ref, o_ref = rest
        else:
            (o_ref,) = rest
        y = jnp.dot(x_ref[...], w_ref[...],
                    preferred_element_type=jnp.float32) + b_ref[...]
        y = _gn_core(y, g_ref[...], be_ref[...], eg_ref[...], egt_ref[...], n)
        if has_sc:
            y = y + sc_ref[...]
        if relu:
            y = _leaky(y)
        o_ref[...] = y

    return _PC(body, out_shape=jax.ShapeDtypeStruct((n, co), jnp.float32))(*args)


def _gnact(x, gamma, beta):
    n, c = x.shape
    eg, egt = _group_mats_np(c)

    def body(x_ref, g_ref, be_ref, eg_ref, egt_ref, o_ref):
        y = _gn_core(x_ref[...], g_ref[...], be_ref[...], eg_ref[...],
                     egt_ref[...], n)
        o_ref[...] = _leaky(y)

    return _PC(body, out_shape=jax.ShapeDtypeStruct((n, c), jnp.float32))(
        x, gamma.reshape(1, -1), beta.reshape(1, -1), jnp.asarray(eg),
        jnp.asarray(egt))


def _mm(x, w, b=None):
    n = x.shape[0]
    co = w.shape[1]
    has_b = b is not None
    args = [x, w] + ([b.reshape(1, -1)] if has_b else [])

    def body(x_ref, w_ref, *rest):
        if has_b:
            b_ref, o_ref = rest
        else:
            (o_ref,) = rest
        y = jnp.dot(x_ref[...], w_ref[...], preferred_element_type=jnp.float32)
        if has_b:
            y = y + b_ref[...]
        o_ref[...] = y

    return _PC(body, out_shape=jax.ShapeDtypeStruct((n, co), jnp.float32))(*args)


def _kpconv_tc(g, qrep, kpt, wmat, sigma, m):
    cp = wmat.shape[1]
    d = wmat.shape[2]
    dt = g.shape[1]
    tm = min(128, m)
    tmk = tm * _K

    def body(g_ref, q_ref, kpt_ref, w_ref, o_ref):
        gg = g_ref[...]
        rel = gg[:, 0:3] - q_ref[...]
        kp = kpt_ref[...]
        sqn = jnp.sum(rel * rel, axis=1, keepdims=True)
        dots = jnp.dot(rel, kp, preferred_element_type=jnp.float32)
        kp2 = jnp.sum(kp * kp, axis=0, keepdims=True)
        sq = sqn + kp2 - 2.0 * dots
        w = jnp.maximum(0.0, 1.0 - jnp.sqrt(sq + 1e-12) / sigma)
        f3 = gg[:, 16:16 + cp].reshape(tm, _K, cp)
        w3 = w.reshape(tm, _K, _KS)
        acc = jnp.zeros((tm, d), jnp.float32)
        for p_i in range(_KS):
            ap = jnp.sum(w3[:, :, p_i:p_i + 1] * f3, axis=1)
            acc = acc + jnp.dot(ap, w_ref[p_i],
                                preferred_element_type=jnp.float32)
        o_ref[...] = acc

    return _PC(
        body,
        grid=(m // tm,),
        in_specs=[
            pl.BlockSpec((tmk, dt), lambda i: (i, 0)),
            pl.BlockSpec((tmk, 3), lambda i: (i, 0)),
            pl.BlockSpec((3, _KS), lambda i: (0, 0)),
            pl.BlockSpec((_KS, cp, d), lambda i: (0, 0, 0)),
        ],
        out_specs=pl.BlockSpec((tm, d), lambda i: (i, 0)),
        out_shape=jax.ShapeDtypeStruct((m, d), jnp.float32),
    )(g, qrep, kpt, wmat)


def _maxred(g, m):
    c = g.shape[1]
    tm = min(128, m)
    tmk = tm * _K

    def body(g_ref, o_ref):
        f3 = g_ref[...].reshape(tm, _K, c)
        o_ref[...] = jnp.max(f3, axis=1)

    return _PC(
        body,
        grid=(m // tm,),
        in_specs=[pl.BlockSpec((tmk, c), lambda i: (i, 0))],
        out_specs=pl.BlockSpec((tm, c), lambda i: (i, 0)),
        out_shape=jax.ShapeDtypeStruct((m, c), jnp.float32),
    )(g)


def _pool_scatter(idx, cvals, nseg):
    n, c = cvals.shape

    def body(idx_ref, c_ref, o_ref):
        o_ref[...] = jnp.full((nseg, c), -jnp.inf, jnp.float32)

        def it(m_i, carry):
            seg = idx_ref[m_i]
            row = c_ref[pl.ds(m_i, 1), :]
            cur = o_ref[pl.ds(seg, 1), :]
            o_ref[pl.ds(seg, 1), :] = jnp.maximum(cur, row)
            return carry

        lax.fori_loop(0, n, it, 0)

    return _PC(
        body,
        in_specs=[
            pl.BlockSpec(memory_space=pltpu.MemorySpace.SMEM),
            pl.BlockSpec(memory_space=pltpu.MemorySpace.VMEM),
        ],
        out_specs=pl.BlockSpec(memory_space=pltpu.MemorySpace.VMEM),
        out_shape=jax.ShapeDtypeStruct((nseg, c), jnp.float32),
    )(idx, cvals)


def _nbr_table(spts, h):
    pts = jnp.concatenate([spts, jnp.full((8, 3), 1e6, jnp.float32)], 0)
    pts16 = jnp.pad(pts, ((0, 0), (0, 13)))
    fp = jnp.concatenate([h, jnp.zeros((8, h.shape[1]), jnp.float32)], 0)
    t = jnp.concatenate([pts16, fp], 1)
    dt = -t.shape[1] % 128
    if dt:
        t = jnp.pad(t, ((0, 0), (0, dt)))
    return t


def _zero_table(x):
    return jnp.concatenate([x, jnp.zeros((8, x.shape[1]), jnp.float32)], 0)


def _qrep(qp):
    m = qp.shape[0]
    return jnp.broadcast_to(qp[:, None, :], (m, _K, 3)).reshape(m * _K, 3)


def _plane_idx(p, reso):
    xy = p[:, 0:2] / (1.0 + 99.0 + 1e-3) + 0.5
    xy = jnp.clip(xy, 0.0, 1.0 - 1e-3)
    xi = (xy * reso).astype(jnp.int32)
    return xi[:, 0] + reso * xi[:, 1]


def _pool(pts_lvl, cvals, reso):
    idx = _plane_idx(pts_lvl, reso)
    plane = _pool_scatter(idx, cvals, reso * reso)
    return _sc_gather(plane, idx)


def _pad_w(kw):
    cp = max(16, kw.shape[1])
    if kw.shape[1] == cp:
        return kw
    return jnp.pad(kw, ((0, 0), (0, cp - kw.shape[1]), (0, 0)))


def _pad_feats(h):
    cp = max(16, h.shape[1])
    if h.shape[1] == cp:
        return h
    return jnp.pad(h, ((0, 0), (0, cp - h.shape[1])))


def _kpconv(h, qp, sp, neigh, kw, kp_pts, sigma):
    g = _sc_gather(_nbr_table(sp, _pad_feats(h)), neigh.reshape(-1))
    return _kpconv_tc(g, _qrep(qp), kp_pts.T, _pad_w(kw), sigma, qp.shape[0])


def _res_block(x, qp, sp, neigh, p, radius, sigma, ku, strided=False):
    h = _mmgn(x, p['u1'], relu=True)
    hp = _kpconv(h, qp, sp, neigh, p['kw'], ku * radius, sigma)
    h2 = _gnact(hp, p['g'], p['be'])
    if strided:
        scg = _sc_gather(_zero_table(x), neigh.reshape(-1))
        scx = _maxred(scg, qp.shape[0])
    else:
        scx = x
    scv = _mmgn(scx, p['sc'], relu=False) if 'sc' in p else scx
    return _mmgn(h2, p['u2'], relu=True, sc=scv)


def kernel(feats, points_0, points_1, points_2, points_3, points_4,
           neighbors_0, neighbors_1, neighbors_2, neighbors_3, neighbors_4,
           subsampling_0, subsampling_1, subsampling_2, subsampling_3,
           upsampling_0, params):
    P = params
    pts = [points_0, points_1, points_2, points_3, points_4]
    neighs = [neighbors_0, neighbors_1, neighbors_2, neighbors_3, neighbors_4]
    subs = [subsampling_0, subsampling_1, subsampling_2, subsampling_3]
    R, S = 2.0, 2.0
    ku = P['kp_unit']

    hp = _kpconv(feats, pts[0], pts[0], neighs[0], P['e11']['kw'], ku * R, S)
    x1 = _gnact(hp, P['e11']['g'], P['e11']['be'])
    x1 = _res_block(x1, pts[0], pts[0], neighs[0], P['e12'], R, S, ku)
    out1 = _mm(x1, P['l1'])
    x1 = jnp.concatenate([x1, _pool(pts[0], out1, _RESO * 8)], 1)

    x2 = _res_block(x1, pts[1], pts[0], subs[0], P['e21'], R, S, ku, True)
    x2 = _res_block(x2, pts[1], pts[1], neighs[1], P['e22'], 2 * R, 2 * S, ku)
    x2 = _res_block(x2, pts[1], pts[1], neighs[1], P['e23'], 2 * R, 2 * S, ku)
    out2 = _mm(x2, P['l2'])
    x2 = jnp.concatenate([x2, _pool(pts[1], out2, _RESO * 4)], 1)

    x3 = _res_block(x2, pts[2], pts[1], subs[1], P['e31'], 2 * R, 2 * S, ku, True)
    x3 = _res_block(x3, pts[2], pts[2], neighs[2], P['e32'], 4 * R, 4 * S, ku)
    x3 = _res_block(x3, pts[2], pts[2], neighs[2], P['e33'], 4 * R, 4 * S, ku)
    out3 = _mm(x3, P['l3'])
    x3 = jnp.concatenate([x3, _pool(pts[2], out3, _RESO * 2)], 1)

    x4 = _res_block(x3, pts[3], pts[2], subs[2], P['e41'], 4 * R, 4 * S, ku, True)
    x4 = _res_block(x4, pts[3], pts[3], neighs[3], P['e42'], 8 * R, 8 * S, ku)
    x4 = _res_block(x4, pts[3], pts[3], neighs[3], P['e43'], 8 * R, 8 * S, ku)
    out4 = _mm(x4, P['l4'])
    x4 = jnp.concatenate([x4, _pool(pts[3], out4, _RESO)], 1)

    x5 = _res_block(x4, pts[4], pts[3], subs[3], P['e51'], 8 * R, 8 * S, ku, True)
    x5 = _res_block(x5, pts[4], pts[4], neighs[4], P['e52'], 16 * R, 16 * S, ku)
    x5 = _res_block(x5, pts[4], pts[4], neighs[4], P['e53'], 16 * R, 16 * S, ku)
    feats_s5_out = _mm(x5, P['l5'])[None]

    up = _sc_gather(_zero_table(x5), upsampling_0[:, 0])
    lat4 = jnp.concatenate([up, x4], 1)
    lat4 = _mmgn(lat4, P['d40'], relu=True)
    lat4 = _mmgn(lat4, P['d41'], relu=True)
    feats_s4_out = _mm(lat4, P['d42w'], P['d42b'])[None]
    return feats_s5_out, feats_s4_out


# speedup vs baseline: 2.4826x; 1.0059x over previous
import functools

import jax
import jax.numpy as jnp
import numpy as np
from jax import lax
from jax.experimental import pallas as pl
from jax.experimental.pallas import tpu as pltpu
from jax.experimental.pallas import tpu_sc as plsc

_LRELU = 0.1
_EPS = 1e-5
_GN = 32
_RESO = 16
_KS = 15
_K = 32

_PC = pl.pallas_call


def _sc_gather(table, idx):
    V, D = table.shape
    B = idx.shape[0]
    info = plsc.get_sparse_core_info()
    nw = info.num_cores * info.num_subcores
    bpw = B // nw
    ch = min(128, bpw, 49152 // D)
    ch = 1 << (ch.bit_length() - 1)
    nch = bpw // ch
    nc = info.num_cores
    mesh = plsc.VectorSubcoreMesh(core_axis_name="c", subcore_axis_name="s")

    @functools.partial(
        pl.kernel,
        mesh=mesh,
        out_type=jax.ShapeDtypeStruct((B, D), jnp.float32),
        scratch_types=[
            pltpu.VMEM((ch,), jnp.int32),
            pltpu.VMEM((ch,), jnp.int32),
            pltpu.VMEM((ch, D), jnp.float32),
            pltpu.VMEM((ch, D), jnp.float32),
            pltpu.SemaphoreType.DMA,
            pltpu.SemaphoreType.DMA,
        ],
    )
    def gk(table_hbm, idx_hbm, out_hbm, idx0, idx1, rows0, rows1, s0, s1):
        wid = lax.axis_index("s") * nc + lax.axis_index("c")
        base = wid * bpw

        def fire(j, idx_v, rows_v, sem):
            off = base + j * ch
            pltpu.sync_copy(idx_hbm.at[pl.ds(off, ch)], idx_v)
            pltpu.async_copy(table_hbm.at[idx_v], rows_v, sem)

        def drain(j, rows_v, sem):
            off = base + j * ch
            pltpu.make_async_copy(table_hbm.at[pl.ds(0, ch)], rows_v,
                                  sem).wait()
            pltpu.sync_copy(rows_v, out_hbm.at[pl.ds(off, ch)])

        if nch == 1:
            fire(0, idx0, rows0, s0)
            drain(0, rows0, s0)
        else:
            fire(0, idx0, rows0, s0)

            def body(i, carry):
                j = 2 * i
                fire(j + 1, idx1, rows1, s1)
                drain(j, rows0, s0)
                fire(j + 2, idx0, rows0, s0)
                drain(j + 1, rows1, s1)
                return carry

            lax.fori_loop(0, nch // 2 - 1, body, 0)
            j = nch - 2
            fire(j + 1, idx1, rows1, s1)
            drain(j, rows0, s0)
            drain(j + 1, rows1, s1)

    return gk(table, idx)


def _leaky(x):
    return jnp.where(x >= 0, x, _LRELU * x)


@functools.lru_cache(maxsize=None)
def _group_mats_np(c):
    cg = c // _GN
    e = np.zeros((c, _GN), np.float32)
    e[np.arange(c), np.arange(c) // cg] = 1.0
    return e, e.T.copy()


def _gn_core(y, gamma, beta, eg, egt, nrows):
    c = y.shape[1]
    cnt = float(nrows * (c // _GN))
    s1 = jnp.sum(y, axis=0, keepdims=True)
    gm = jnp.dot(s1, eg, preferred_element_type=jnp.float32) / cnt
    mean = jnp.dot(gm, egt, preferred_element_type=jnp.float32)
    y0 = y - mean
    s2 = jnp.sum(y0 * y0, axis=0, keepdims=True)
    gv = jnp.dot(s2, eg, preferred_element_type=jnp.float32) / cnt
    var = jnp.dot(gv, egt, preferred_element_type=jnp.float32)
    return y0 * lax.rsqrt(var + _EPS) * gamma + beta


def _mmgn(x, p, relu=True, sc=None, pre=None):
    n, ci = x.shape
    co = p['w'].shape[1]
    eg, egt = _group_mats_np(co)
    args = [x, p['w'], p['b'].reshape(1, -1), p['g'].reshape(1, -1),
            p['be'].reshape(1, -1), jnp.asarray(eg), jnp.asarray(egt)]
    has_sc = sc is not None
    if has_sc:
        args.append(sc)
    has_pre = pre is not None
    if has_pre:
        peg, pegt = _group_mats_np(ci)
        args += [pre[0].reshape(1, -1), pre[1].reshape(1, -1),
                 jnp.asarray(peg), jnp.asarray(pegt)]

    def body(x_ref, w_ref, b_ref, g_ref, be_ref, eg_ref, egt_ref, *rest):
        rest = list(rest)
        sc_ref = rest.pop(0) if has_sc else None
        if has_pre:
            pg_ref, pbe_ref, peg_ref, pegt_ref = rest[:4]
            rest = rest[4:]
        (o_ref,) = rest
        xv = x_ref[...]
        if has_pre:
            xv = _leaky(_gn_core(xv, pg_ref[...], pbe_ref[...], peg_ref[...],
                                 pegt_ref[...], n))
        y = jnp.dot(xv, w_ref[...],
                    preferred_element_type=jnp.float32) + b_ref[...]
        y = _gn_core(y, g_ref[...], be_ref[...], eg_ref[...], egt_ref[...], n)
        if has_sc:
            y = y + sc_ref[...]
        if relu:
            y = _leaky(y)
        o_ref[...] = y

    return _PC(body, out_shape=jax.ShapeDtypeStruct((n, co), jnp.float32))(*args)


def _gnact(x, gamma, beta):
    n, c = x.shape
    eg, egt = _group_mats_np(c)

    def body(x_ref, g_ref, be_ref, eg_ref, egt_ref, o_ref):
        y = _gn_core(x_ref[...], g_ref[...], be_ref[...], eg_ref[...],
                     egt_ref[...], n)
        o_ref[...] = _leaky(y)

    return _PC(body, out_shape=jax.ShapeDtypeStruct((n, c), jnp.float32))(
        x, gamma.reshape(1, -1), beta.reshape(1, -1), jnp.asarray(eg),
        jnp.asarray(egt))


def _mm(x, w, b=None):
    n = x.shape[0]
    co = w.shape[1]
    has_b = b is not None
    args = [x, w] + ([b.reshape(1, -1)] if has_b else [])

    def body(x_ref, w_ref, *rest):
        if has_b:
            b_ref, o_ref = rest
        else:
            (o_ref,) = rest
        y = jnp.dot(x_ref[...], w_ref[...], preferred_element_type=jnp.float32)
        if has_b:
            y = y + b_ref[...]
        o_ref[...] = y

    return _PC(body, out_shape=jax.ShapeDtypeStruct((n, co), jnp.float32))(*args)


def _kpconv_tc(g, qrep, kpt, wmat, sigma, m):
    cp = wmat.shape[1]
    d = wmat.shape[2]
    dt = g.shape[1]
    tm = min(128, m)
    tmk = tm * _K

    def body(g_ref, q_ref, kpt_ref, w_ref, o_ref):
        gg = g_ref[...]
        rel = gg[:, 0:3] - q_ref[...]
        kp = kpt_ref[...]
        sqn = jnp.sum(rel * rel, axis=1, keepdims=True)
        dots = jnp.dot(rel, kp, preferred_element_type=jnp.float32)
        kp2 = jnp.sum(kp * kp, axis=0, keepdims=True)
        sq = sqn + kp2 - 2.0 * dots
        w = jnp.maximum(0.0, 1.0 - jnp.sqrt(sq + 1e-12) / sigma)
        f3 = gg[:, 16:16 + cp].reshape(tm, _K, cp)
        w3 = w.reshape(tm, _K, _KS)
        acc = jnp.zeros((tm, d), jnp.float32)
        for p_i in range(_KS):
            ap = jnp.sum(w3[:, :, p_i:p_i + 1] * f3, axis=1)
            acc = acc + jnp.dot(ap, w_ref[p_i],
                                preferred_element_type=jnp.float32)
        o_ref[...] = acc

    return _PC(
        body,
        grid=(m // tm,),
        in_specs=[
            pl.BlockSpec((tmk, dt), lambda i: (i, 0)),
            pl.BlockSpec((tmk, 3), lambda i: (i, 0)),
            pl.BlockSpec((3, _KS), lambda i: (0, 0)),
            pl.BlockSpec((_KS, cp, d), lambda i: (0, 0, 0)),
        ],
        out_specs=pl.BlockSpec((tm, d), lambda i: (i, 0)),
        out_shape=jax.ShapeDtypeStruct((m, d), jnp.float32),
    )(g, qrep, kpt, wmat)


def _maxred(g, m):
    c = g.shape[1]
    tm = min(128, m)
    tmk = tm * _K

    def body(g_ref, o_ref):
        f3 = g_ref[...].reshape(tm, _K, c)
        o_ref[...] = jnp.max(f3, axis=1)

    return _PC(
        body,
        grid=(m // tm,),
        in_specs=[pl.BlockSpec((tmk, c), lambda i: (i, 0))],
        out_specs=pl.BlockSpec((tm, c), lambda i: (i, 0)),
        out_shape=jax.ShapeDtypeStruct((m, c), jnp.float32),
    )(g)


def _pool_scatter(idx, cvals, nseg):
    n, c = cvals.shape

    def body(idx_ref, c_ref, o_ref, p1, p2, p3):
        ninf = jnp.full((nseg, c), -jnp.inf, jnp.float32)
        o_ref[...] = ninf
        p1[...] = ninf
        p2[...] = ninf
        p3[...] = ninf
        subs = (o_ref, p1, p2, p3)

        def it(i, carry):
            for s, ref in enumerate(subs):
                m_i = 4 * i + s
                seg = idx_ref[m_i]
                row = c_ref[pl.ds(m_i, 1), :]
                cur = ref[pl.ds(seg, 1), :]
                ref[pl.ds(seg, 1), :] = jnp.maximum(cur, row)
            return carry

        lax.fori_loop(0, n // 4, it, 0)
        o_ref[...] = jnp.maximum(jnp.maximum(o_ref[...], p1[...]),
                                 jnp.maximum(p2[...], p3[...]))

    return _PC(
        body,
        in_specs=[
            pl.BlockSpec(memory_space=pltpu.MemorySpace.SMEM),
            pl.BlockSpec(memory_space=pltpu.MemorySpace.VMEM),
        ],
        out_specs=pl.BlockSpec(memory_space=pltpu.MemorySpace.VMEM),
        out_shape=jax.ShapeDtypeStruct((nseg, c), jnp.float32),
        scratch_shapes=[pltpu.VMEM((nseg, c), jnp.float32)] * 3,
    )(idx, cvals)


def _nbr_table(spts, h):
    pts = jnp.concatenate([spts, jnp.full((8, 3), 1e6, jnp.float32)], 0)
    pts16 = jnp.pad(pts, ((0, 0), (0, 13)))
    fp = jnp.concatenate([h, jnp.zeros((8, h.shape[1]), jnp.float32)], 0)
    t = jnp.concatenate([pts16, fp], 1)
    dt = -t.shape[1] % 128
    if dt:
        t = jnp.pad(t, ((0, 0), (0, dt)))
    return t


def _zero_table(x):
    return jnp.concatenate([x, jnp.zeros((8, x.shape[1]), jnp.float32)], 0)


def _qrep(qp):
    m = qp.shape[0]
    return jnp.broadcast_to(qp[:, None, :], (m, _K, 3)).reshape(m * _K, 3)


def _plane_idx(p, reso):
    xy = p[:, 0:2] / (1.0 + 99.0 + 1e-3) + 0.5
    xy = jnp.clip(xy, 0.0, 1.0 - 1e-3)
    xi = (xy * reso).astype(jnp.int32)
    return xi[:, 0] + reso * xi[:, 1]


def _pool(pts_lvl, cvals, reso):
    idx = _plane_idx(pts_lvl, reso)
    plane = _pool_scatter(idx, cvals, reso * reso)
    return _sc_gather(plane, idx)


def _pad_w(kw):
    cp = max(16, kw.shape[1])
    if kw.shape[1] == cp:
        return kw
    return jnp.pad(kw, ((0, 0), (0, cp - kw.shape[1]), (0, 0)))


def _pad_feats(h):
    cp = max(16, h.shape[1])
    if h.shape[1] == cp:
        return h
    return jnp.pad(h, ((0, 0), (0, cp - h.shape[1])))


def _kpconv(h, qp, sp, neigh, kw, kp_pts, sigma):
    g = _sc_gather(_nbr_table(sp, _pad_feats(h)), neigh.reshape(-1))
    return _kpconv_tc(g, _qrep(qp), kp_pts.T, _pad_w(kw), sigma, qp.shape[0])


def _res_block(x, qp, sp, neigh, p, radius, sigma, ku, strided=False):
    h = _mmgn(x, p['u1'], relu=True)
    hp = _kpconv(h, qp, sp, neigh, p['kw'], ku * radius, sigma)
    if strided:
        scg = _sc_gather(_zero_table(x), neigh.reshape(-1))
        scx = _maxred(scg, qp.shape[0])
    else:
        scx = x
    scv = _mmgn(scx, p['sc'], relu=False) if 'sc' in p else scx
    return _mmgn(hp, p['u2'], relu=True, sc=scv, pre=(p['g'], p['be']))


def kernel(feats, points_0, points_1, points_2, points_3, points_4,
           neighbors_0, neighbors_1, neighbors_2, neighbors_3, neighbors_4,
           subsampling_0, subsampling_1, subsampling_2, subsampling_3,
           upsampling_0, params):
    P = params
    pts = [points_0, points_1, points_2, points_3, points_4]
    neighs = [neighbors_0, neighbors_1, neighbors_2, neighbors_3, neighbors_4]
    subs = [subsampling_0, subsampling_1, subsampling_2, subsampling_3]
    R, S = 2.0, 2.0
    ku = P['kp_unit']

    hp = _kpconv(feats, pts[0], pts[0], neighs[0], P['e11']['kw'], ku * R, S)
    x1 = _gnact(hp, P['e11']['g'], P['e11']['be'])
    x1 = _res_block(x1, pts[0], pts[0], neighs[0], P['e12'], R, S, ku)
    out1 = _mm(x1, P['l1'])
    x1 = jnp.concatenate([x1, _pool(pts[0], out1, _RESO * 8)], 1)

    x2 = _res_block(x1, pts[1], pts[0], subs[0], P['e21'], R, S, ku, True)
    x2 = _res_block(x2, pts[1], pts[1], neighs[1], P['e22'], 2 * R, 2 * S, ku)
    x2 = _res_block(x2, pts[1], pts[1], neighs[1], P['e23'], 2 * R, 2 * S, ku)
    out2 = _mm(x2, P['l2'])
    x2 = jnp.concatenate([x2, _pool(pts[1], out2, _RESO * 4)], 1)

    x3 = _res_block(x2, pts[2], pts[1], subs[1], P['e31'], 2 * R, 2 * S, ku, True)
    x3 = _res_block(x3, pts[2], pts[2], neighs[2], P['e32'], 4 * R, 4 * S, ku)
    x3 = _res_block(x3, pts[2], pts[2], neighs[2], P['e33'], 4 * R, 4 * S, ku)
    out3 = _mm(x3, P['l3'])
    x3 = jnp.concatenate([x3, _pool(pts[2], out3, _RESO * 2)], 1)

    x4 = _res_block(x3, pts[3], pts[2], subs[2], P['e41'], 4 * R, 4 * S, ku, True)
    x4 = _res_block(x4, pts[3], pts[3], neighs[3], P['e42'], 8 * R, 8 * S, ku)
    x4 = _res_block(x4, pts[3], pts[3], neighs[3], P['e43'], 8 * R, 8 * S, ku)
    out4 = _mm(x4, P['l4'])
    x4 = jnp.concatenate([x4, _pool(pts[3], out4, _RESO)], 1)

    x5 = _res_block(x4, pts[4], pts[3], subs[3], P['e51'], 8 * R, 8 * S, ku, True)
    x5 = _res_block(x5, pts[4], pts[4], neighs[4], P['e52'], 16 * R, 16 * S, ku)
    x5 = _res_block(x5, pts[4], pts[4], neighs[4], P['e53'], 16 * R, 16 * S, ku)
    feats_s5_out = _mm(x5, P['l5'])[None]

    up = _sc_gather(_zero_table(x5), upsampling_0[:, 0])
    lat4 = jnp.concatenate([up, x4], 1)
    lat4 = _mmgn(lat4, P['d40'], relu=True)
    lat4 = _mmgn(lat4, P['d41'], relu=True)
    feats_s4_out = _mm(lat4, P['d42w'], P['d42b'])[None]
    return feats_s5_out, feats_s4_out
